# baseline (speedup 1.0000x reference)
import functools

import jax
import jax.numpy as jnp
from jax import lax
from jax.experimental import pallas as pl
from jax.experimental.pallas import tpu as pltpu
from jax.experimental.pallas import tpu_sc as plsc

BATCH = 16384
HIST_LEN = 50
EMB_DIM = 64

NUM_CORES = 2
NUM_SUBCORES = 16
NUM_WORKERS = NUM_CORES * NUM_SUBCORES

BPW = BATCH // NUM_WORKERS
BBLK = 128
NBB = BPW // BBLK
NCH = HIST_LEN * NBB
NBUF = 2

def _gather_body(x_hbm, table_hbm, out_hbm,
                 idx_v, idxc0, idxc1, rows0, rows1, tr0, tr1,
                 gsem0, gsem1, osem0, osem1):
    wid = lax.axis_index("s") * NUM_CORES + lax.axis_index("c")
    b0 = wid * BPW

    pltpu.sync_copy(x_hbm.at[pl.ds(b0, BPW)], idx_v)

    i16 = lax.iota(jnp.int32, 16)
    r16 = [i16 + 16 * k for k in range(8)]

    slots = ((idxc0, rows0, tr0, gsem0, osem0),
             (idxc1, rows1, tr1, gsem1, osem1))

    def chunk_hb(j):
        h = j // NBB
        bb = j - h * NBB
        return h, bb

    def build_idx(j, s):
        idxc, _, _, _, _ = s
        h, bb = chunk_hb(j)
        col = jnp.full((16,), 0, jnp.int32) + h
        base = bb * BBLK
        for k in range(8):
            v = plsc.load_gather(idx_v, [r16[k] + base, col])
            idxc[pl.ds(16 * k, 16)] = v

    def gstart(s):
        idxc, rows, _, gsem, _ = s
        pltpu.make_async_copy(table_hbm.at[idxc], rows, gsem).start()

    def gwait(s):
        idxc, rows, _, gsem, _ = s
        pltpu.make_async_copy(table_hbm.at[idxc], rows, gsem).wait()

    def out_ref(j):
        h, bb = chunk_hb(j)
        return out_hbm.at[pl.ds(h, 1), :, pl.ds(wid * NBB + bb, 1)]

    def ostart(j, s):
        _, _, tr, _, osem = s
        pltpu.make_async_copy(
            tr.at[:, :, :, :, pl.ds(0, BBLK)], out_ref(j), osem).start()

    def owait(j, s):
        _, _, tr, _, osem = s
        pltpu.make_async_copy(
            tr.at[:, :, :, :, pl.ds(0, BBLK)], out_ref(j), osem).wait()

    zero16 = i16 * 0
    dblk16 = [lax.shift_right_logical(i16 + 16 * jj, 3) for jj in range(4)]
    dsub16 = [lax.bitwise_and(i16 + 16 * jj, 7) for jj in range(4)]

    def transpose(s):
        _, rows, tr, _, _ = s
        for r0 in range(0, BBLK, 4):
            cols = [jnp.full((16,), r0 + i, jnp.int32) for i in range(4)]
            vs = [rows[r0 + i, pl.ds(16 * jj, 16)]
                  for i in range(4) for jj in range(4)]
            for i in range(4):
                for jj in range(4):
                    plsc.store_scatter(
                        tr,
                        [zero16, dblk16[jj], zero16, dsub16[jj], cols[i]],
                        vs[i * 4 + jj])

    for b in range(NBUF):
        build_idx(b, slots[b])
        gstart(slots[b])

    def round_body(r, carry):
        for b in range(NBUF):
            j = r * NBUF + b
            s = slots[b]
            gwait(s)

            @pl.when(r > 0)
            def _():
                owait(j - NBUF, s)

            transpose(s)
            ostart(j, s)
            jn = j + NBUF

            @pl.when(jn < NCH)
            def _():
                build_idx(jn, s)
                gstart(s)

        return carry

    lax.fori_loop(0, NCH // NBUF, round_body, 0)

    for b in range(NBUF):
        owait(NCH - NBUF + b, slots[b])


@jax.jit
def _embedding_gather(x, table):
    mesh = plsc.VectorSubcoreMesh(
        core_axis_name="c", subcore_axis_name="s",
        num_cores=NUM_CORES, num_subcores=NUM_SUBCORES,
    )
    run = functools.partial(
        pl.kernel,
        out_type=jax.ShapeDtypeStruct(
            (HIST_LEN, 8, BATCH // BBLK, 8, BBLK), jnp.float32),
        mesh=mesh,
        scratch_types=[
            pltpu.VMEM((BPW, HIST_LEN), jnp.int32),
            pltpu.VMEM((BBLK,), jnp.int32),
            pltpu.VMEM((BBLK,), jnp.int32),
            pltpu.VMEM((BBLK, EMB_DIM), jnp.float32),
            pltpu.VMEM((BBLK, EMB_DIM), jnp.float32),
            pltpu.VMEM((1, 8, 1, 8, BBLK + 1), jnp.float32),
            pltpu.VMEM((1, 8, 1, 8, BBLK + 1), jnp.float32),
            pltpu.SemaphoreType.DMA,
            pltpu.SemaphoreType.DMA,
            pltpu.SemaphoreType.DMA,
            pltpu.SemaphoreType.DMA,
        ],
        compiler_params=pltpu.CompilerParams(
            use_tc_tiling_on_sc=False, needs_layout_passes=False),
    )(_gather_body)
    return run(x, table)


def kernel(x, table):
    out5 = _embedding_gather(x.astype(jnp.int32), table)
    return jnp.transpose(out5, (2, 4, 0, 1, 3)).reshape(
        BATCH, HIST_LEN, EMB_DIM)


# speedup vs baseline: 1.0367x; 1.0367x over previous
import functools

import jax
import jax.numpy as jnp
from jax import lax
from jax.experimental import pallas as pl
from jax.experimental.pallas import tpu as pltpu
from jax.experimental.pallas import tpu_sc as plsc

BATCH = 16384
HIST_LEN = 50
EMB_DIM = 64

NUM_CORES = 2
NUM_SUBCORES = 16
NUM_WORKERS = NUM_CORES * NUM_SUBCORES

BPW = BATCH // NUM_WORKERS
BBLK = 128
NBB = BPW // BBLK
NCH = HIST_LEN * NBB
NBUF = 2

def _gather_body(x_hbm, table_hbm, out_hbm,
                 idx_v, idxc0, idxc1, rows0, rows1, tr0, tr1,
                 gsem0, gsem1, osem0, osem1):
    wid = lax.axis_index("s") * NUM_CORES + lax.axis_index("c")
    b0 = wid * BPW

    pltpu.sync_copy(x_hbm.at[pl.ds(b0, BPW)], idx_v)

    i16 = lax.iota(jnp.int32, 16)
    r16 = [i16 + 16 * k for k in range(8)]

    slots = ((idxc0, rows0, tr0, gsem0, osem0),
             (idxc1, rows1, tr1, gsem1, osem1))

    def chunk_hb(j):
        h = j // NBB
        bb = j - h * NBB
        return h, bb

    def build_idx(j, s):
        idxc, _, _, _, _ = s
        h, bb = chunk_hb(j)
        col = jnp.full((16,), 0, jnp.int32) + h
        base = bb * BBLK
        for k in range(8):
            v = plsc.load_gather(idx_v, [r16[k] + base, col])
            idxc[pl.ds(16 * k, 16)] = v

    def gstart(s):
        idxc, rows, _, gsem, _ = s
        pltpu.make_async_copy(table_hbm.at[idxc], rows, gsem).start()

    def gwait(s):
        idxc, rows, _, gsem, _ = s
        pltpu.make_async_copy(table_hbm.at[idxc], rows, gsem).wait()

    def out_ref(j):
        h, bb = chunk_hb(j)
        return out_hbm.at[pl.ds(h, 1), :, pl.ds(wid * NBB + bb, 1)]

    def ostart(j, s):
        _, _, tr, _, osem = s
        pltpu.make_async_copy(
            tr.at[:, :, :, :, pl.ds(0, BBLK)], out_ref(j), osem).start()

    def owait(j, s):
        _, _, tr, _, osem = s
        pltpu.make_async_copy(
            tr.at[:, :, :, :, pl.ds(0, BBLK)], out_ref(j), osem).wait()

    zero16 = i16 * 0
    dblk16 = [lax.shift_right_logical(i16 + 16 * jj, 3) for jj in range(4)]
    dsub16 = [lax.bitwise_and(i16 + 16 * jj, 7) for jj in range(4)]

    def transpose(s):
        _, rows, tr, _, _ = s
        for r0 in range(0, BBLK, 2):
            cols = [jnp.full((16,), r0 + i, jnp.int32) for i in range(2)]
            vs = [rows[r0 + i, pl.ds(16 * jj, 16)]
                  for i in range(2) for jj in range(4)]
            for i in range(2):
                for jj in range(4):
                    plsc.store_scatter(
                        tr,
                        [zero16, dblk16[jj], zero16, dsub16[jj], cols[i]],
                        vs[i * 4 + jj])

    for b in range(NBUF):
        build_idx(b, slots[b])
        gstart(slots[b])

    def round_body(r, carry):
        for b in range(NBUF):
            j = r * NBUF + b
            s = slots[b]
            gwait(s)

            @pl.when(r > 0)
            def _():
                owait(j - NBUF, s)

            transpose(s)
            ostart(j, s)
            jn = j + NBUF

            @pl.when(jn < NCH)
            def _():
                build_idx(jn, s)
                gstart(s)

        return carry

    lax.fori_loop(0, NCH // NBUF, round_body, 0)

    for b in range(NBUF):
        owait(NCH - NBUF + b, slots[b])


@jax.jit
def _embedding_gather(x, table):
    mesh = plsc.VectorSubcoreMesh(
        core_axis_name="c", subcore_axis_name="s",
        num_cores=NUM_CORES, num_subcores=NUM_SUBCORES,
    )
    run = functools.partial(
        pl.kernel,
        out_type=jax.ShapeDtypeStruct(
            (HIST_LEN, 8, BATCH // BBLK, 8, BBLK), jnp.float32),
        mesh=mesh,
        scratch_types=[
            pltpu.VMEM((BPW, HIST_LEN), jnp.int32),
            pltpu.VMEM((BBLK,), jnp.int32),
            pltpu.VMEM((BBLK,), jnp.int32),
            pltpu.VMEM((BBLK, EMB_DIM), jnp.float32),
            pltpu.VMEM((BBLK, EMB_DIM), jnp.float32),
            pltpu.VMEM((1, 8, 1, 8, BBLK + 1), jnp.float32),
            pltpu.VMEM((1, 8, 1, 8, BBLK + 1), jnp.float32),
            pltpu.SemaphoreType.DMA,
            pltpu.SemaphoreType.DMA,
            pltpu.SemaphoreType.DMA,
            pltpu.SemaphoreType.DMA,
        ],
        compiler_params=pltpu.CompilerParams(
            use_tc_tiling_on_sc=False, needs_layout_passes=False),
    )(_gather_body)
    return run(x, table)


def kernel(x, table):
    out5 = _embedding_gather(x.astype(jnp.int32), table)
    return jnp.transpose(out5, (2, 4, 0, 1, 3)).reshape(
        BATCH, HIST_LEN, EMB_DIM)
